# initial kernel scaffold (unmeasured)
import jax
import jax.numpy as jnp
from jax import lax
from jax.experimental import pallas as pl
from jax.experimental.pallas import tpu as pltpu

N_DEV = 4
SQ = 512
D = 1024
SKV = 2048
HQ = 8
HKV = 2
DH = 128
SCALE = 0.08838834764831843


def _body(x_ref, wq_ref, wo_ref, k_ref, v_ref, out_ref,
          o_comm, ml_comm, o_send, o_recv, ml_send, ml_recv):
    my = lax.axis_index("i")

    xb = x_ref[...].astype(jnp.bfloat16)

    O = [None] * HQ
    M = [None] * HQ
    L = [None] * HQ
    for g in range(HKV):
        kg = k_ref[:, g * DH:(g + 1) * DH].astype(jnp.bfloat16)
        vg = v_ref[:, g * DH:(g + 1) * DH].astype(jnp.bfloat16)
        for h in range(g * 4, g * 4 + 4):
            wq_h = wq_ref[:, h * DH:(h + 1) * DH].astype(jnp.bfloat16)
            q = jnp.dot(xb, wq_h, preferred_element_type=jnp.float32)
            qb = (q * SCALE).astype(jnp.bfloat16)
            s = lax.dot_general(
                qb, kg, (((1,), (1,)), ((), ())),
                preferred_element_type=jnp.float32)
            m = jnp.max(s, axis=1, keepdims=True)
            p = jnp.exp(s - m)
            l = jnp.sum(p, axis=1, keepdims=True)
            o = jnp.dot(p.astype(jnp.bfloat16), vg,
                        preferred_element_type=jnp.float32)
            O[h], M[h], L[h] = o, m, l
            o_comm[0, :, h * DH:(h + 1) * DH] = o.astype(jnp.bfloat16)
            ml_comm[0, :, h:h + 1] = m
            ml_comm[0, :, HQ + h:HQ + h + 1] = l

    sends = []
    for off in (1, 2, 3):
        dst = lax.rem(my + off, N_DEV)
        slot = N_DEV - off
        for buf, ssem, rsem in ((o_comm, o_send, o_recv),
                                (ml_comm, ml_send, ml_recv)):
            r = pltpu.make_async_remote_copy(
                src_ref=buf.at[0],
                dst_ref=buf.at[slot],
                send_sem=ssem.at[off],
                recv_sem=rsem.at[slot],
                device_id=(dst,),
                device_id_type=pl.DeviceIdType.MESH,
            )
            r.start()
            sends.append(r)

    for j in (1, 2, 3):
        for buf, ssem, rsem in ((o_comm, o_send, o_recv),
                                (ml_comm, ml_send, ml_recv)):
            r = pltpu.make_async_remote_copy(
                src_ref=buf.at[0],
                dst_ref=buf.at[j],
                send_sem=ssem.at[0],
                recv_sem=rsem.at[j],
                device_id=(my,),
                device_id_type=pl.DeviceIdType.MESH,
            )
            r.wait_recv()
        for h in range(HQ):
            m_j = ml_comm[j, :, h:h + 1]
            l_j = ml_comm[j, :, HQ + h:HQ + h + 1]
            o_j = o_comm[j, :, h * DH:(h + 1) * DH].astype(jnp.float32)
            m_new = jnp.maximum(M[h], m_j)
            a = jnp.exp(M[h] - m_new)
            b = jnp.exp(m_j - m_new)
            O[h] = O[h] * a + o_j * b
            L[h] = L[h] * a + l_j * b
            M[h] = m_new

    for r in sends:
        r.wait_send()

    acc = jnp.zeros((SQ, D), jnp.float32)
    for h in range(HQ):
        on = (O[h] / L[h]).astype(jnp.bfloat16)
        wo_h = wo_ref[h * DH:(h + 1) * DH, :].astype(jnp.bfloat16)
        acc = acc + jnp.dot(on, wo_h, preferred_element_type=jnp.float32)
    out_ref[...] = acc


def kernel(x, Wq, Wo, K_ext, V_ext):
    x2 = x.reshape(SQ, D)
    k2 = K_ext.reshape(SKV, HKV * DH)
    v2 = V_ext.reshape(SKV, HKV * DH)
    out = pl.pallas_call(
        _body,
        out_shape=jax.ShapeDtypeStruct((SQ, D), jnp.float32),
        in_specs=[pl.BlockSpec(memory_space=pltpu.VMEM)] * 5,
        out_specs=pl.BlockSpec(memory_space=pltpu.VMEM),
        scratch_shapes=[
            pltpu.VMEM((N_DEV, SQ, HQ * DH), jnp.bfloat16),
            pltpu.VMEM((N_DEV, SQ, 2 * HQ), jnp.float32),
            pltpu.SemaphoreType.DMA((N_DEV,)),
            pltpu.SemaphoreType.DMA((N_DEV,)),
            pltpu.SemaphoreType.DMA((N_DEV,)),
            pltpu.SemaphoreType.DMA((N_DEV,)),
        ],
        compiler_params=pltpu.CompilerParams(collective_id=0),
    )(x2, Wq, Wo, k2, v2)
    return out.reshape(1, SQ, D)


# baseline (device time: 76435 ns/iter reference)
import jax
import jax.numpy as jnp
from jax import lax
from jax.experimental import pallas as pl
from jax.experimental.pallas import tpu as pltpu

N_DEV = 4
SQ = 512
D = 1024
SKV = 2048
HQ = 8
HKV = 2
DH = 128
SCALE = 0.08838834764831843


def _body(x_ref, wq_ref, wo_ref, k_ref, v_ref, out_ref,
          o_comm, ml_comm, o_send, o_recv, ml_send, ml_recv):
    my = lax.axis_index("i")

    xb = x_ref[...].astype(jnp.bfloat16)

    O = [None] * HQ
    M = [None] * HQ
    L = [None] * HQ
    for g in range(HKV):
        kg = k_ref[:, g * DH:(g + 1) * DH].astype(jnp.bfloat16)
        vg = v_ref[:, g * DH:(g + 1) * DH].astype(jnp.bfloat16)
        for h in range(g * 4, g * 4 + 4):
            wq_h = wq_ref[:, h * DH:(h + 1) * DH].astype(jnp.bfloat16)
            q = jnp.dot(xb, wq_h, preferred_element_type=jnp.float32)
            qb = (q * SCALE).astype(jnp.bfloat16)
            s = lax.dot_general(
                qb, kg, (((1,), (1,)), ((), ())),
                preferred_element_type=jnp.float32)
            m = jnp.max(s, axis=1, keepdims=True)
            p = jnp.exp(s - m)
            l = jnp.sum(p, axis=1, keepdims=True)
            o = jnp.dot(p.astype(jnp.bfloat16), vg,
                        preferred_element_type=jnp.float32)
            O[h], M[h], L[h] = o, m, l
            o_comm[0, :, h * DH:(h + 1) * DH] = o.astype(jnp.bfloat16)
            ml_comm[0, :, h:h + 1] = m
            ml_comm[0, :, HQ + h:HQ + h + 1] = l

    sends = []
    for off in (1, 2, 3):
        dst = lax.rem(my + off, N_DEV)
        slot = N_DEV - off
        for buf, ssem, rsem in ((o_comm, o_send, o_recv),
                                (ml_comm, ml_send, ml_recv)):
            r = pltpu.make_async_remote_copy(
                src_ref=buf.at[0],
                dst_ref=buf.at[slot],
                send_sem=ssem.at[off],
                recv_sem=rsem.at[slot],
                device_id=(dst,),
                device_id_type=pl.DeviceIdType.MESH,
            )
            r.start()
            sends.append(r)

    for j in (1, 2, 3):
        for buf, ssem, rsem in ((o_comm, o_send, o_recv),
                                (ml_comm, ml_send, ml_recv)):
            r = pltpu.make_async_remote_copy(
                src_ref=buf.at[0],
                dst_ref=buf.at[j],
                send_sem=ssem.at[0],
                recv_sem=rsem.at[j],
                device_id=(my,),
                device_id_type=pl.DeviceIdType.MESH,
            )
            r.wait_recv()
        for h in range(HQ):
            m_j = ml_comm[j, :, h:h + 1]
            l_j = ml_comm[j, :, HQ + h:HQ + h + 1]
            o_j = o_comm[j, :, h * DH:(h + 1) * DH].astype(jnp.float32)
            m_new = jnp.maximum(M[h], m_j)
            a = jnp.exp(M[h] - m_new)
            b = jnp.exp(m_j - m_new)
            O[h] = O[h] * a + o_j * b
            L[h] = L[h] * a + l_j * b
            M[h] = m_new

    for r in sends:
        r.wait_send()

    acc = jnp.zeros((SQ, D), jnp.float32)
    for h in range(HQ):
        on = (O[h] / L[h]).astype(jnp.bfloat16)
        wo_h = wo_ref[h * DH:(h + 1) * DH, :].astype(jnp.bfloat16)
        acc = acc + jnp.dot(on, wo_h, preferred_element_type=jnp.float32)
    out_ref[...] = acc


def kernel(x, Wq, Wo, K_ext, V_ext):
    x2 = x.reshape(SQ, D)
    k2 = K_ext.reshape(SKV, HKV * DH)
    v2 = V_ext.reshape(SKV, HKV * DH)
    out = pl.pallas_call(
        _body,
        out_shape=jax.ShapeDtypeStruct((SQ, D), jnp.float32),
        in_specs=[pl.BlockSpec(memory_space=pltpu.VMEM)] * 5,
        out_specs=pl.BlockSpec(memory_space=pltpu.VMEM),
        scratch_shapes=[
            pltpu.VMEM((N_DEV, SQ, HQ * DH), jnp.bfloat16),
            pltpu.VMEM((N_DEV, SQ, 2 * HQ), jnp.float32),
            pltpu.SemaphoreType.DMA((N_DEV,)),
            pltpu.SemaphoreType.DMA((N_DEV,)),
            pltpu.SemaphoreType.DMA((N_DEV,)),
            pltpu.SemaphoreType.DMA((N_DEV,)),
        ],
        compiler_params=pltpu.CompilerParams(
            vmem_limit_bytes=100 * 1024 * 1024,
        ),
    )(x2, Wq, Wo, k2, v2)
    return out.reshape(1, SQ, D)


# device time: 50097 ns/iter; 1.5257x vs baseline; 1.5257x over previous
import jax
import jax.numpy as jnp
from jax import lax
from jax.experimental import pallas as pl
from jax.experimental.pallas import tpu as pltpu

N_DEV = 4
SQ = 512
QPD = SQ // N_DEV
D = 1024
SKV = 2048
HQ = 8
HKV = 2
HPG = HQ // HKV
DH = 128
SCALE = 0.08838834764831843


def _body(x_ref, wq_ref, wo_ref, k_ref, v_ref, out_ref,
          o_loc, ml_loc, rs_o, rs_ml,
          rs_o_send, rs_o_recv, rs_ml_send, rs_ml_recv,
          ag_send, ag_recv):
    my = lax.axis_index("i")

    xb = x_ref[...].astype(jnp.bfloat16)

    sends = []

    for g in range(HKV):
        kg = k_ref[:, g * DH:(g + 1) * DH].astype(jnp.bfloat16)
        vg = v_ref[:, g * DH:(g + 1) * DH].astype(jnp.bfloat16)
        for hh in range(HPG):
            h = g * HPG + hh
            wq_h = wq_ref[:, h * DH:(h + 1) * DH].astype(jnp.bfloat16)
            q = jnp.dot(xb, wq_h, preferred_element_type=jnp.float32)
            qb = (q * SCALE).astype(jnp.bfloat16)
            s = lax.dot_general(
                qb, kg, (((1,), (1,)), ((), ())),
                preferred_element_type=jnp.float32)
            m = jnp.max(s, axis=1, keepdims=True)
            p = jnp.exp(s - m)
            l = jnp.sum(p, axis=1, keepdims=True)
            o = jnp.dot(p.astype(jnp.bfloat16), vg,
                        preferred_element_type=jnp.float32)
            o_loc[g, :, hh * DH:(hh + 1) * DH] = o.astype(jnp.bfloat16)
            ml_loc[g, :, hh:hh + 1] = m
            ml_loc[g, :, HPG + hh:HPG + hh + 1] = l

        for off in (1, 2, 3):
            dst = lax.rem(my + off, N_DEV)
            slot = N_DEV - off
            for src_buf, dst_buf, ssem, rsem in (
                    (o_loc, rs_o, rs_o_send, rs_o_recv),
                    (ml_loc, rs_ml, rs_ml_send, rs_ml_recv)):
                r = pltpu.make_async_remote_copy(
                    src_ref=src_buf.at[g, pl.ds(dst * QPD, QPD), :],
                    dst_ref=dst_buf.at[g, slot],
                    send_sem=ssem.at[g, off],
                    recv_sem=rsem.at[g, slot],
                    device_id=(dst,),
                    device_id_type=pl.DeviceIdType.MESH,
                )
                r.start()
                sends.append(r)

    O = [None] * HQ
    M = [None] * HQ
    L = [None] * HQ
    mine = pl.ds(my * QPD, QPD)
    for g in range(HKV):
        for hh in range(HPG):
            h = g * HPG + hh
            O[h] = o_loc[g, mine, hh * DH:(hh + 1) * DH].astype(jnp.float32)
            M[h] = ml_loc[g, mine, hh:hh + 1]
            L[h] = ml_loc[g, mine, HPG + hh:HPG + hh + 1]
        for j in (1, 2, 3):
            for src_buf, dst_buf, ssem, rsem in (
                    (o_loc, rs_o, rs_o_send, rs_o_recv),
                    (ml_loc, rs_ml, rs_ml_send, rs_ml_recv)):
                r = pltpu.make_async_remote_copy(
                    src_ref=src_buf.at[g, pl.ds(0, QPD), :],
                    dst_ref=dst_buf.at[g, j],
                    send_sem=ssem.at[g, 0],
                    recv_sem=rsem.at[g, j],
                    device_id=(my,),
                    device_id_type=pl.DeviceIdType.MESH,
                )
                r.wait_recv()
            for hh in range(HPG):
                h = g * HPG + hh
                m_j = rs_ml[g, j, :, hh:hh + 1]
                l_j = rs_ml[g, j, :, HPG + hh:HPG + hh + 1]
                o_j = rs_o[g, j, :, hh * DH:(hh + 1) * DH].astype(jnp.float32)
                m_new = jnp.maximum(M[h], m_j)
                a = jnp.exp(M[h] - m_new)
                b = jnp.exp(m_j - m_new)
                O[h] = O[h] * a + o_j * b
                L[h] = L[h] * a + l_j * b
                M[h] = m_new

    acc = jnp.zeros((QPD, D), jnp.float32)
    for h in range(HQ):
        on = (O[h] / L[h]).astype(jnp.bfloat16)
        wo_h = wo_ref[h * DH:(h + 1) * DH, :].astype(jnp.bfloat16)
        acc = acc + jnp.dot(on, wo_h, preferred_element_type=jnp.float32)
    out_ref[mine, :] = acc.astype(jnp.bfloat16)

    for off in (1, 2, 3):
        dst = lax.rem(my + off, N_DEV)
        r = pltpu.make_async_remote_copy(
            src_ref=out_ref.at[mine, :],
            dst_ref=out_ref.at[mine, :],
            send_sem=ag_send.at[off],
            recv_sem=ag_recv.at[N_DEV - off],
            device_id=(dst,),
            device_id_type=pl.DeviceIdType.MESH,
        )
        r.start()
        sends.append(r)
    for j in (1, 2, 3):
        owner = lax.rem(my + (N_DEV - j), N_DEV)
        r = pltpu.make_async_remote_copy(
            src_ref=out_ref.at[mine, :],
            dst_ref=out_ref.at[pl.ds(owner * QPD, QPD), :],
            send_sem=ag_send.at[0],
            recv_sem=ag_recv.at[j],
            device_id=(my,),
            device_id_type=pl.DeviceIdType.MESH,
        )
        r.wait_recv()

    for r in sends:
        r.wait_send()


def kernel(x, Wq, Wo, K_ext, V_ext):
    x2 = x.reshape(SQ, D)
    k2 = K_ext.reshape(SKV, HKV * DH)
    v2 = V_ext.reshape(SKV, HKV * DH)
    out = pl.pallas_call(
        _body,
        out_shape=jax.ShapeDtypeStruct((SQ, D), jnp.bfloat16),
        in_specs=[pl.BlockSpec(memory_space=pltpu.VMEM)] * 5,
        out_specs=pl.BlockSpec(memory_space=pltpu.VMEM),
        scratch_shapes=[
            pltpu.VMEM((HKV, SQ, HPG * DH), jnp.bfloat16),
            pltpu.VMEM((HKV, SQ, 2 * HPG), jnp.float32),
            pltpu.VMEM((HKV, N_DEV, QPD, HPG * DH), jnp.bfloat16),
            pltpu.VMEM((HKV, N_DEV, QPD, 2 * HPG), jnp.float32),
            pltpu.SemaphoreType.DMA((HKV, N_DEV)),
            pltpu.SemaphoreType.DMA((HKV, N_DEV)),
            pltpu.SemaphoreType.DMA((HKV, N_DEV)),
            pltpu.SemaphoreType.DMA((HKV, N_DEV)),
            pltpu.SemaphoreType.DMA((N_DEV,)),
            pltpu.SemaphoreType.DMA((N_DEV,)),
        ],
        compiler_params=pltpu.CompilerParams(
            vmem_limit_bytes=100 * 1024 * 1024,
        ),
    )(x2, Wq, Wo, k2, v2)
    return out.reshape(1, SQ, D)


# device time: 47949 ns/iter; 1.5941x vs baseline; 1.0448x over previous
import jax
import jax.numpy as jnp
from jax import lax
from jax.experimental import pallas as pl
from jax.experimental.pallas import tpu as pltpu

N_DEV = 4
SQ = 512
QPD = SQ // N_DEV
D = 1024
SKV = 2048
HQ = 8
HKV = 2
HPG = HQ // HKV
DH = 128
SCALE = 0.08838834764831843


def _body(x_ref, wq_ref, wo_ref, k_ref, v_ref, out_ref,
          o_loc, ml_loc, rs_o, rs_ml,
          rs_o_send, rs_o_recv, rs_ml_send, rs_ml_recv,
          ag_send, ag_recv):
    my = lax.axis_index("i")

    xb = x_ref[...].astype(jnp.bfloat16)
    q_all = (jnp.dot(xb, wq_ref[...].astype(jnp.bfloat16),
                     preferred_element_type=jnp.float32)
             * SCALE).astype(jnp.bfloat16)

    sends = []

    for g in range(HKV):
        kg = k_ref[:, g * DH:(g + 1) * DH].astype(jnp.bfloat16)
        vg = v_ref[:, g * DH:(g + 1) * DH].astype(jnp.bfloat16)
        for hh in range(HPG):
            h = g * HPG + hh
            qb = q_all[:, h * DH:(h + 1) * DH]
            s = lax.dot_general(
                qb, kg, (((1,), (1,)), ((), ())),
                preferred_element_type=jnp.float32)
            m = jnp.max(s, axis=1, keepdims=True)
            p = jnp.exp(s - m)
            l = jnp.sum(p, axis=1, keepdims=True)
            o = jnp.dot(p.astype(jnp.bfloat16), vg,
                        preferred_element_type=jnp.float32)
            o_loc[g, :, hh * DH:(hh + 1) * DH] = o.astype(jnp.bfloat16)
            ml_loc[g, :, hh:hh + 1] = m
            ml_loc[g, :, HPG + hh:HPG + hh + 1] = l

        for off in (1, 2, 3):
            dst = lax.rem(my + off, N_DEV)
            slot = N_DEV - off
            for src_buf, dst_buf, ssem, rsem in (
                    (o_loc, rs_o, rs_o_send, rs_o_recv),
                    (ml_loc, rs_ml, rs_ml_send, rs_ml_recv)):
                r = pltpu.make_async_remote_copy(
                    src_ref=src_buf.at[g, pl.ds(dst * QPD, QPD), :],
                    dst_ref=dst_buf.at[g, slot],
                    send_sem=ssem.at[g, off],
                    recv_sem=rsem.at[g, slot],
                    device_id=(dst,),
                    device_id_type=pl.DeviceIdType.MESH,
                )
                r.start()
                sends.append(r)

    O = [None] * HQ
    M = [None] * HQ
    L = [None] * HQ
    mine = pl.ds(my * QPD, QPD)
    for g in range(HKV):
        for hh in range(HPG):
            h = g * HPG + hh
            O[h] = o_loc[g, mine, hh * DH:(hh + 1) * DH].astype(jnp.float32)
            M[h] = ml_loc[g, mine, hh:hh + 1]
            L[h] = ml_loc[g, mine, HPG + hh:HPG + hh + 1]
        for j in (1, 2, 3):
            for src_buf, dst_buf, ssem, rsem in (
                    (o_loc, rs_o, rs_o_send, rs_o_recv),
                    (ml_loc, rs_ml, rs_ml_send, rs_ml_recv)):
                r = pltpu.make_async_remote_copy(
                    src_ref=src_buf.at[g, pl.ds(0, QPD), :],
                    dst_ref=dst_buf.at[g, j],
                    send_sem=ssem.at[g, 0],
                    recv_sem=rsem.at[g, j],
                    device_id=(my,),
                    device_id_type=pl.DeviceIdType.MESH,
                )
                r.wait_recv()
            for hh in range(HPG):
                h = g * HPG + hh
                m_j = rs_ml[g, j, :, hh:hh + 1]
                l_j = rs_ml[g, j, :, HPG + hh:HPG + hh + 1]
                o_j = rs_o[g, j, :, hh * DH:(hh + 1) * DH].astype(jnp.float32)
                m_new = jnp.maximum(M[h], m_j)
                a = jnp.exp(M[h] - m_new)
                b = jnp.exp(m_j - m_new)
                O[h] = O[h] * a + o_j * b
                L[h] = L[h] * a + l_j * b
                M[h] = m_new

    on_all = jnp.concatenate(
        [(O[h] / L[h]).astype(jnp.bfloat16) for h in range(HQ)],
        axis=1)
    acc = jnp.dot(on_all, wo_ref[...].astype(jnp.bfloat16),
                  preferred_element_type=jnp.float32)
    out_ref[mine, :] = acc.astype(jnp.bfloat16)

    for off in (1, 2, 3):
        dst = lax.rem(my + off, N_DEV)
        r = pltpu.make_async_remote_copy(
            src_ref=out_ref.at[mine, :],
            dst_ref=out_ref.at[mine, :],
            send_sem=ag_send.at[off],
            recv_sem=ag_recv.at[N_DEV - off],
            device_id=(dst,),
            device_id_type=pl.DeviceIdType.MESH,
        )
        r.start()
        sends.append(r)
    for j in (1, 2, 3):
        owner = lax.rem(my + (N_DEV - j), N_DEV)
        r = pltpu.make_async_remote_copy(
            src_ref=out_ref.at[mine, :],
            dst_ref=out_ref.at[pl.ds(owner * QPD, QPD), :],
            send_sem=ag_send.at[0],
            recv_sem=ag_recv.at[j],
            device_id=(my,),
            device_id_type=pl.DeviceIdType.MESH,
        )
        r.wait_recv()

    for r in sends:
        r.wait_send()


def kernel(x, Wq, Wo, K_ext, V_ext):
    x2 = x.reshape(SQ, D)
    k2 = K_ext.reshape(SKV, HKV * DH)
    v2 = V_ext.reshape(SKV, HKV * DH)
    out = pl.pallas_call(
        _body,
        out_shape=jax.ShapeDtypeStruct((SQ, D), jnp.bfloat16),
        in_specs=[pl.BlockSpec(memory_space=pltpu.VMEM)] * 5,
        out_specs=pl.BlockSpec(memory_space=pltpu.VMEM),
        scratch_shapes=[
            pltpu.VMEM((HKV, SQ, HPG * DH), jnp.bfloat16),
            pltpu.VMEM((HKV, SQ, 2 * HPG), jnp.float32),
            pltpu.VMEM((HKV, N_DEV, QPD, HPG * DH), jnp.bfloat16),
            pltpu.VMEM((HKV, N_DEV, QPD, 2 * HPG), jnp.float32),
            pltpu.SemaphoreType.DMA((HKV, N_DEV)),
            pltpu.SemaphoreType.DMA((HKV, N_DEV)),
            pltpu.SemaphoreType.DMA((HKV, N_DEV)),
            pltpu.SemaphoreType.DMA((HKV, N_DEV)),
            pltpu.SemaphoreType.DMA((N_DEV,)),
            pltpu.SemaphoreType.DMA((N_DEV,)),
        ],
        compiler_params=pltpu.CompilerParams(
            vmem_limit_bytes=100 * 1024 * 1024,
        ),
    )(x2, Wq, Wo, k2, v2)
    return out.reshape(1, SQ, D)
